# Initial kernel scaffold; baseline (speedup 1.0000x reference)
import jax
import jax.numpy as jnp
from jax.experimental import pallas as pl


def kernel(inputs, w, v, b):
    raise NotImplementedError("write your pallas kernel here")


# speedup vs baseline: 11.2966x; 11.2966x over previous
import functools

import jax
import jax.numpy as jnp
from jax import lax
from jax.experimental import pallas as pl
from jax.experimental.pallas import tpu as pltpu
from jax.experimental.pallas import tpu_sc as plsc

_F = 26
_K = 16
_D = (_F - 1) * _K
_B = 4096
_VOCAB = 100000

_info = plsc.get_sparse_core_info()
_NC, _NS = _info.num_cores, _info.num_subcores
_NW = _NC * _NS
_BPW = _B // _NW
_NBUF = 4


def _pairs():
    out = []
    for i in range(_F - 1):
        for j in range(i + 1, _F):
            out.append((i, j))
    return out

_PAIRS = _pairs()


def _ffm_body(idx2d_hbm, w_hbm, v_hbm, b_hbm, out_hbm,
              idx2d, tot, out_v, b_v, *rest):
    rows = list(rest[:_NBUF])
    wrows = list(rest[_NBUF:2 * _NBUF])
    sems = list(rest[2 * _NBUF:3 * _NBUF])
    wsems = list(rest[3 * _NBUF:4 * _NBUF])

    wid = lax.axis_index("s") * _NC + lax.axis_index("c")
    base = wid * _BPW

    pltpu.sync_copy(idx2d_hbm.at[pl.ds(base, _BPW)], idx2d)
    pltpu.sync_copy(b_hbm, b_v)

    def start(item, par):
        pltpu.make_async_copy(v_hbm.at[idx2d.at[item]], rows[par],
                              sems[par]).start()
        pltpu.make_async_copy(w_hbm.at[idx2d.at[item]],
                              wrows[par].at[pl.ds(0, _F)],
                              wsems[par]).start()

    def wait(item, par):
        pltpu.make_async_copy(v_hbm.at[idx2d.at[item]], rows[par],
                              sems[par]).wait()
        pltpu.make_async_copy(w_hbm.at[idx2d.at[item]],
                              wrows[par].at[pl.ds(0, _F)],
                              wsems[par]).wait()

    for par in range(_NBUF):
        start(par, par)

    iota = lax.iota(jnp.int32, 16)
    tail_mask = iota < (_F - 16)

    def item_body(g, carry):
        for par in range(_NBUF):
            item = g * _NBUF + par
            wait(item, par)
            r = rows[par]
            accs = [None] * 8
            for t, (i, j) in enumerate(_PAIRS):
                a = r[i, pl.ds((j - 1) * _K, _K)]
                bb = r[j, pl.ds(i * _K, _K)]
                m = t % 8
                prod = a * bb
                accs[m] = prod if accs[m] is None else accs[m] + prod
            wa = wrows[par][pl.ds(0, 16)]
            wb = wrows[par][pl.ds(16, 16)]
            accs[0] = accs[0] + wa
            accs[1] = accs[1] + jnp.where(tail_mask, wb, 0.0)
            acc = (((accs[0] + accs[1]) + (accs[2] + accs[3]))
                   + ((accs[4] + accs[5]) + (accs[6] + accs[7])))
            s1 = acc + lax.rev(acc, (0,))
            total = (((s1[0] + s1[1]) + (s1[2] + s1[3]))
                     + ((s1[4] + s1[5]) + (s1[6] + s1[7])))
            tot[item, :] = jnp.full((16,), total, jnp.float32)

            @pl.when(item + _NBUF < _BPW)
            def _():
                start(item + _NBUF, par)
        return carry

    lax.fori_loop(0, _BPW // _NBUF, item_body, None)

    bvec = b_v[...]
    for blk in range(_BPW // 16):
        y = tot[blk * 16, :]
        for lane in range(1, 16):
            y = jnp.where(iota == lane, tot[blk * 16 + lane, :], y)
        out_v[pl.ds(blk * 16, 16)] = 1.0 / (1.0 + jnp.exp(-(y + bvec)))
    pltpu.sync_copy(out_v, out_hbm.at[pl.ds(base, _BPW)])


@jax.jit
def _ffm(inputs2d, w1, v2, b16):
    mesh = plsc.VectorSubcoreMesh(core_axis_name="c", subcore_axis_name="s")
    scratch = [
        pltpu.VMEM((_BPW, _F), jnp.int32),
        pltpu.VMEM((_BPW, 16), jnp.float32),
        pltpu.VMEM((_BPW,), jnp.float32),
        pltpu.VMEM((16,), jnp.float32),
    ]
    scratch += [pltpu.VMEM((_F, _D), jnp.float32) for _ in range(_NBUF)]
    scratch += [pltpu.VMEM((32,), jnp.float32) for _ in range(_NBUF)]
    scratch += [pltpu.SemaphoreType.DMA for _ in range(2 * _NBUF)]
    kfn = functools.partial(
        pl.kernel,
        mesh=mesh,
        out_type=jax.ShapeDtypeStruct((_B,), jnp.float32),
        scratch_types=scratch,
        compiler_params=pltpu.CompilerParams(use_tc_tiling_on_sc=False),
    )(_ffm_body)
    return kfn(inputs2d, w1, v2, b16)


def kernel(inputs, w, v, b):
    v2 = v.reshape(_VOCAB, _D)
    w1 = w.reshape(_VOCAB)
    b16 = jnp.broadcast_to(b.astype(jnp.float32), (16,))
    return _ffm(inputs, w1, v2, b16)
